# Initial kernel scaffold; baseline (speedup 1.0000x reference)
import jax
import jax.numpy as jnp
from jax.experimental import pallas as pl


def kernel(y_pred, y_true):
    raise NotImplementedError("write your pallas kernel here")


# speedup vs baseline: 2.7464x; 2.7464x over previous
import jax
import jax.numpy as jnp
from jax.experimental import pallas as pl
from jax.experimental.pallas import tpu as pltpu

_CLS = 512
_N = 50000
_EPS = 1e-07
_BLK = 1000


def _body(yp_ref, yt_ref, out_ref, cnt_ref, cor_ref):
    i = pl.program_id(0)

    @pl.when(i == 0)
    def _init():
        cnt_ref[...] = jnp.zeros_like(cnt_ref)
        cor_ref[...] = jnp.zeros_like(cor_ref)

    v = yp_ref[...]
    m = jnp.max(v, axis=1, keepdims=True)
    eqf = (v == m).astype(jnp.float32)
    iota = jax.lax.broadcasted_iota(jnp.int32, v.shape, 1)
    teqf = (iota == yt_ref[...]).astype(jnp.float32)
    cnt_ref[...] += jnp.sum(eqf, axis=0, keepdims=True)
    cor_ref[...] += jnp.sum(eqf * teqf, axis=0, keepdims=True)

    @pl.when(i == pl.num_programs(0) - 1)
    def _fin():
        ratio = cor_ref[...] / (cnt_ref[...] + _EPS)
        out_ref[...] = jnp.sum(ratio, axis=1, keepdims=True) / _CLS


def kernel(y_pred, y_true):
    yt = y_true.astype(jnp.int32).reshape(_N, 1)
    out = pl.pallas_call(
        _body,
        grid=(_N // _BLK,),
        in_specs=[
            pl.BlockSpec((_BLK, _CLS), lambda i: (i, 0)),
            pl.BlockSpec((_BLK, 1), lambda i: (i, 0)),
        ],
        out_specs=pl.BlockSpec((1, 1), lambda i: (0, 0)),
        out_shape=jax.ShapeDtypeStruct((1, 1), jnp.float32),
        scratch_shapes=[
            pltpu.VMEM((1, _CLS), jnp.float32),
            pltpu.VMEM((1, _CLS), jnp.float32),
        ],
    )(y_pred, yt)
    return out[0, 0]


# speedup vs baseline: 3.2868x; 1.1968x over previous
import jax
import jax.numpy as jnp
from jax.experimental import pallas as pl
from jax.experimental.pallas import tpu as pltpu

_CLS = 512
_N = 50000
_EPS = 1e-07
_BLK = 2000


def _body(yp_ref, yt_ref, out_ref, cnt_ref, cor_ref):
    i = pl.program_id(0)

    @pl.when(i == 0)
    def _init():
        cnt_ref[...] = jnp.zeros_like(cnt_ref)
        cor_ref[...] = jnp.zeros_like(cor_ref)

    v = yp_ref[...]
    m = jnp.max(v, axis=1, keepdims=True)
    eqf = (v == m).astype(jnp.float32)
    iota = jax.lax.broadcasted_iota(jnp.int32, v.shape, 1)
    teqf = (iota == yt_ref[...]).astype(jnp.float32)
    cnt_ref[...] += jnp.sum(eqf, axis=0, keepdims=True)
    cor_ref[...] += jnp.sum(eqf * teqf, axis=0, keepdims=True)

    @pl.when(i == pl.num_programs(0) - 1)
    def _fin():
        ratio = cor_ref[...] / (cnt_ref[...] + _EPS)
        out_ref[...] = jnp.sum(ratio, axis=1, keepdims=True) / _CLS


def kernel(y_pred, y_true):
    yt = y_true.astype(jnp.int32).reshape(_N, 1)
    out = pl.pallas_call(
        _body,
        grid=(_N // _BLK,),
        in_specs=[
            pl.BlockSpec((_BLK, _CLS), lambda i: (i, 0)),
            pl.BlockSpec((_BLK, 1), lambda i: (i, 0)),
        ],
        out_specs=pl.BlockSpec((1, 1), lambda i: (0, 0)),
        out_shape=jax.ShapeDtypeStruct((1, 1), jnp.float32),
        scratch_shapes=[
            pltpu.VMEM((1, _CLS), jnp.float32),
            pltpu.VMEM((1, _CLS), jnp.float32),
        ],
    )(y_pred, yt)
    return out[0, 0]


# speedup vs baseline: 3.6674x; 1.1158x over previous
import jax
import jax.numpy as jnp
from jax.experimental import pallas as pl
from jax.experimental.pallas import tpu as pltpu

_CLS = 512
_N = 50000
_EPS = 1e-07
_BLK = 5000


def _body(yp_ref, yt_ref, out_ref, cnt_ref, cor_ref):
    i = pl.program_id(0)

    @pl.when(i == 0)
    def _init():
        cnt_ref[...] = jnp.zeros_like(cnt_ref)
        cor_ref[...] = jnp.zeros_like(cor_ref)

    v = yp_ref[...]
    m = jnp.max(v, axis=1, keepdims=True)
    eqf = (v == m).astype(jnp.float32)
    iota = jax.lax.broadcasted_iota(jnp.int32, v.shape, 1)
    teqf = (iota == yt_ref[...]).astype(jnp.float32)
    cnt_ref[...] += jnp.sum(eqf, axis=0, keepdims=True)
    cor_ref[...] += jnp.sum(eqf * teqf, axis=0, keepdims=True)

    @pl.when(i == pl.num_programs(0) - 1)
    def _fin():
        ratio = cor_ref[...] / (cnt_ref[...] + _EPS)
        out_ref[...] = jnp.sum(ratio, axis=1, keepdims=True) / _CLS


def kernel(y_pred, y_true):
    yt = y_true.astype(jnp.int32).reshape(_N, 1)
    out = pl.pallas_call(
        _body,
        grid=(_N // _BLK,),
        in_specs=[
            pl.BlockSpec((_BLK, _CLS), lambda i: (i, 0)),
            pl.BlockSpec((_BLK, 1), lambda i: (i, 0)),
        ],
        out_specs=pl.BlockSpec((1, 1), lambda i: (0, 0)),
        out_shape=jax.ShapeDtypeStruct((1, 1), jnp.float32),
        scratch_shapes=[
            pltpu.VMEM((1, _CLS), jnp.float32),
            pltpu.VMEM((1, _CLS), jnp.float32),
        ],
    )(y_pred, yt)
    return out[0, 0]
